# baseline (speedup 1.0000x reference)
import jax
import jax.numpy as jnp
from jax.experimental import pallas as pl
from jax.experimental.pallas import tpu as pltpu

_B, _N, _W, _R, _C = 64, 1024, 64, 4, 512
_LOUT = 256
_IFACE = _R * _W + _R + _W + 1 + 2 * _W + _R + 2 + 3 * _R


def _controller_body(xr_ref, h_ref, c_ref, wih_ref, whh_ref, bias_ref,
                     wif_ref, bif_ref, ctrl_ref, cnew_ref, iface_ref):
    gates = (jnp.dot(xr_ref[...], wih_ref[...], preferred_element_type=jnp.float32)
             + jnp.dot(h_ref[...], whh_ref[...], preferred_element_type=jnp.float32)
             + bias_ref[...])
    i_g = gates[:, 0 * _C:1 * _C]
    f_g = gates[:, 1 * _C:2 * _C]
    g_g = gates[:, 2 * _C:3 * _C]
    o_g = gates[:, 3 * _C:4 * _C]
    c_new = jax.nn.sigmoid(f_g) * c_ref[...] + jax.nn.sigmoid(i_g) * jnp.tanh(g_g)
    ctrl = jax.nn.sigmoid(o_g) * jnp.tanh(c_new)
    cnew_ref[...] = c_new
    ctrl_ref[...] = ctrl
    iface_ref[...] = (jnp.dot(ctrl, wif_ref[...], preferred_element_type=jnp.float32)
                      + bif_ref[...])


def _mega_body(mem_ref, link_ref, rw_ref, usage_ref, wwts_ref, prec_ref,
               rk_ref, rbeta_ref, wk_ref, wbeta_ref, ev_ref, wv_ref,
               fg_ref, ag_ref, wg_ref, m0_ref, m1_ref, m2_ref,
               linknew_ref, memnew_ref, rwnew_ref, ww_ref, uu_ref,
               precnew_ref, reads_ref):
    mem = mem_ref[0]
    rw = rw_ref[0]
    u = usage_ref[0]
    wwt = wwts_ref[0]
    prec = prec_ref[0]

    f_sig = jax.nn.sigmoid(fg_ref[0])
    t = 1.0 - f_sig * rw
    ret = t[0:1]
    for r in range(1, _R):
        ret = ret * t[r:r + 1]
    uu = (u + wwt - u * wwt) * ret
    uu_ref[0] = uu

    uu_col = jnp.transpose(uu)
    logu = jnp.log(jnp.maximum(uu, 1e-38))
    logu_hi = logu.astype(jnp.bfloat16).astype(jnp.float32)
    logu_lo = logu - logu_hi
    ch = 128
    tri = (jax.lax.broadcasted_iota(jnp.int32, (ch, ch), 1)
           < jax.lax.broadcasted_iota(jnp.int32, (ch, ch), 0))
    cdims1 = (((1,), (1,)), ((), ()))
    s_parts = []
    for s in range(0, _N, ch):
        uu_c = uu_col[s:s + ch]
        pieces = []
        if s > 0:
            pieces.append(jnp.where(uu[:, :s] <= uu_c, 1.0, 0.0))
        ud = uu[:, s:s + ch]
        cond_d = (ud < uu_c) | ((ud == uu_c) & tri)
        pieces.append(jnp.where(cond_d, 1.0, 0.0))
        if s + ch < _N:
            pieces.append(jnp.where(uu[:, s + ch:] < uu_c, 1.0, 0.0))
        cmask = jnp.concatenate(pieces, axis=1) if len(pieces) > 1 else pieces[0]
        lsum = (jax.lax.dot_general(cmask, logu_hi, cdims1,
                                    preferred_element_type=jnp.float32)
                + jax.lax.dot_general(cmask, logu_lo, cdims1,
                                      preferred_element_type=jnp.float32))
        s_parts.append(lsum)
    p = jnp.exp(jnp.concatenate(s_parts, axis=0))
    alloc_col = (1.0 - uu_col) * uu_col * p
    alloc = jnp.transpose(alloc_col)

    ones_w = jnp.ones((1, _W), jnp.float32)
    cdims = (((1,), (1,)), ((), ()))
    mem_nsq = jax.lax.dot_general(ones_w, mem * mem, cdims,
                                  preferred_element_type=jnp.float32)
    mem_norm = jnp.sqrt(mem_nsq) + 1e-8
    wk = wk_ref[0]
    wk_n = wk / (jnp.sqrt(jnp.sum(wk * wk, axis=1, keepdims=True)) + 1e-8)
    beta_w = 1.0 + jax.nn.softplus(wbeta_ref[0])
    logits_w = jax.lax.dot_general(wk_n, mem, cdims,
                                   preferred_element_type=jnp.float32) / mem_norm * beta_w
    logits_w = logits_w - jnp.max(logits_w, axis=1, keepdims=True)
    e_w = jnp.exp(logits_w)
    w_c = e_w / jnp.sum(e_w, axis=1, keepdims=True)

    wg = jax.nn.sigmoid(wg_ref[0])
    ag = jax.nn.sigmoid(ag_ref[0])
    ww = wg * (ag * alloc + (1.0 - ag) * w_c)
    ww_ref[0] = ww
    precnew_ref[0] = (1.0 - jnp.sum(ww, axis=1, keepdims=True)) * prec + ww
    ww_col = jnp.transpose(ww)

    offdiag = (jax.lax.broadcasted_iota(jnp.int32, (128, 128), 0)
               != jax.lax.broadcasted_iota(jnp.int32, (128, 128), 1))
    one_minus_wwc = 1.0 - ww_col
    for a in range(0, _N, 128):
        lchunk = link_ref[0, a:a + 128, :]
        wwc = ww_col[a:a + 128]
        cand = (one_minus_wwc[a:a + 128] - ww) * lchunk + wwc * prec
        linknew_ref[0, a:a + 128, :] = cand
        blk = jnp.where(offdiag, cand[:, a:a + 128], 0.0)
        linknew_ref[0, a:a + 128, a:a + 128] = blk
    lnk = linknew_ref[0]
    fwd = jax.lax.dot_general(rw, lnk, (((1,), (1,)), ((), ())),
                              preferred_element_type=jnp.float32)
    bwd = jax.lax.dot_general(rw, lnk, (((1,), (0,)), ((), ())),
                              preferred_element_type=jnp.float32)

    ev = jax.nn.sigmoid(ev_ref[0])
    wv = wv_ref[0]
    memnew = mem * (1.0 - ww_col * ev) + ww_col * wv
    memnew_ref[0] = memnew
    mn = memnew_ref[0]

    mn_nsq = jax.lax.dot_general(ones_w, mn * mn, cdims,
                                 preferred_element_type=jnp.float32)
    mn_norm = jnp.sqrt(mn_nsq) + 1e-8
    rk = rk_ref[0]
    rk_n = rk / (jnp.sqrt(jnp.sum(rk * rk, axis=1, keepdims=True)) + 1e-8)
    beta_r = 1.0 + jax.nn.softplus(rbeta_ref[0])
    logits_r = jax.lax.dot_general(rk_n, mn, cdims,
                                   preferred_element_type=jnp.float32) / mn_norm * beta_r
    logits_r = logits_r - jnp.max(logits_r, axis=1, keepdims=True)
    e_r = jnp.exp(logits_r)
    r_c = e_r / jnp.sum(e_r, axis=1, keepdims=True)

    m0 = m0_ref[0]
    m1 = m1_ref[0]
    m2 = m2_ref[0]
    mx = jnp.maximum(m0, jnp.maximum(m1, m2))
    e0 = jnp.exp(m0 - mx)
    e1 = jnp.exp(m1 - mx)
    e2 = jnp.exp(m2 - mx)
    es = e0 + e1 + e2
    rwnew = (e0 / es) * bwd + (e1 / es) * r_c + (e2 / es) * fwd
    rwnew_ref[0] = rwnew
    reads_ref[0] = jax.lax.dot_general(rwnew, mn, (((1,), (0,)), ((), ())),
                                       preferred_element_type=jnp.float32)


def _outproj_body(co_ref, wout_ref, bout_ref, out_ref):
    out_ref[...] = (jnp.dot(co_ref[...], wout_ref[...],
                            preferred_element_type=jnp.float32) + bout_ref[...])


def kernel(x, read, h, c, memory, link, r_weights, w_weights, usage,
           precedence, W_ih, W_hh, b_ih, b_hh, W_if, b_if, W_out, b_out):
    B, N, W = memory.shape
    R = r_weights.shape[1]
    C = h.shape[1]

    xr = jnp.concatenate([x, read], axis=1)
    bias = (b_ih + b_hh).reshape(1, 4 * C)
    ctrl, c_new, iface = pl.pallas_call(
        _controller_body,
        out_shape=[
            jax.ShapeDtypeStruct((B, C), jnp.float32),
            jax.ShapeDtypeStruct((B, C), jnp.float32),
            jax.ShapeDtypeStruct((B, _IFACE), jnp.float32),
        ],
        name="dnc_controller",
    )(xr, h, c, W_ih.T, W_hh.T, bias, W_if.T, b_if.reshape(1, _IFACE))

    o = R * W
    r_keys = iface[:, :o].reshape(B, R, W)
    r_betas = iface[:, o:o + R].reshape(B, R, 1); o += R
    w_key = iface[:, o:o + W].reshape(B, 1, W); o += W
    w_beta = iface[:, o:o + 1].reshape(B, 1, 1); o += 1
    e_vec = iface[:, o:o + W].reshape(B, 1, W); o += W
    w_vec = iface[:, o:o + W].reshape(B, 1, W); o += W
    f_gates = iface[:, o:o + R].reshape(B, R, 1); o += R
    a_gate = iface[:, o:o + 1].reshape(B, 1, 1); o += 1
    w_gate = iface[:, o:o + 1].reshape(B, 1, 1); o += 1
    r_modes = iface[:, o:o + 3 * R].reshape(B, R, 3)
    m0 = r_modes[:, :, 0:1]
    m1 = r_modes[:, :, 1:2]
    m2 = r_modes[:, :, 2:3]

    usage3 = usage.reshape(B, 1, N)
    wwts3 = w_weights.reshape(B, 1, N)
    prec3 = precedence.reshape(B, 1, N)

    def _b3(shape):
        return pl.BlockSpec((1,) + shape, lambda b: (b, 0, 0))

    (link_new, mem_new, rw_new, ww3, uu3, precnew3, reads3) = pl.pallas_call(
        _mega_body,
        grid=(B,),
        in_specs=[
            _b3((N, W)),
            _b3((N, N)),
            _b3((R, N)),
            _b3((1, N)),
            _b3((1, N)),
            _b3((1, N)),
            _b3((R, W)),
            _b3((R, 1)),
            _b3((1, W)),
            _b3((1, 1)),
            _b3((1, W)),
            _b3((1, W)),
            _b3((R, 1)),
            _b3((1, 1)),
            _b3((1, 1)),
            _b3((R, 1)),
            _b3((R, 1)),
            _b3((R, 1)),
        ],
        out_specs=[
            _b3((N, N)),
            _b3((N, W)),
            _b3((R, N)),
            _b3((1, N)),
            _b3((1, N)),
            _b3((1, N)),
            _b3((R, W)),
        ],
        out_shape=[
            jax.ShapeDtypeStruct((B, N, N), jnp.float32),
            jax.ShapeDtypeStruct((B, N, W), jnp.float32),
            jax.ShapeDtypeStruct((B, R, N), jnp.float32),
            jax.ShapeDtypeStruct((B, 1, N), jnp.float32),
            jax.ShapeDtypeStruct((B, 1, N), jnp.float32),
            jax.ShapeDtypeStruct((B, 1, N), jnp.float32),
            jax.ShapeDtypeStruct((B, R, W), jnp.float32),
        ],
        compiler_params=pltpu.CompilerParams(
            dimension_semantics=("arbitrary",),
            vmem_limit_bytes=52 * 1024 * 1024,
        ),
        name="dnc_mega",
    )(memory, link, r_weights, usage3, wwts3, prec3, r_keys, r_betas,
      w_key, w_beta, e_vec, w_vec, f_gates, a_gate, w_gate, m0, m1, m2)

    reads_flat = reads3.reshape(B, R * W)

    co = jnp.concatenate([ctrl, reads_flat], axis=1)
    out = pl.pallas_call(
        _outproj_body,
        out_shape=jax.ShapeDtypeStruct((B, _LOUT), jnp.float32),
        name="dnc_outproj",
    )(co, W_out.T, b_out.reshape(1, _LOUT))

    return (out, reads_flat, mem_new, link_new, rw_new,
            ww3.reshape(B, N), uu3.reshape(B, N), precnew3.reshape(B, N),
            ctrl, c_new)


# speedup vs baseline: 1.0472x; 1.0472x over previous
import jax
import jax.numpy as jnp
from jax.experimental import pallas as pl
from jax.experimental.pallas import tpu as pltpu

_B, _N, _W, _R, _C = 64, 1024, 64, 4, 512
_LOUT = 256
_IFACE = _R * _W + _R + _W + 1 + 2 * _W + _R + 2 + 3 * _R


def _controller_body(xr_ref, h_ref, c_ref, wih_ref, whh_ref, bias_ref,
                     wif_ref, bif_ref, ctrl_ref, cnew_ref, iface_ref):
    gates = (jnp.dot(xr_ref[...], wih_ref[...], preferred_element_type=jnp.float32)
             + jnp.dot(h_ref[...], whh_ref[...], preferred_element_type=jnp.float32)
             + bias_ref[...])
    i_g = gates[:, 0 * _C:1 * _C]
    f_g = gates[:, 1 * _C:2 * _C]
    g_g = gates[:, 2 * _C:3 * _C]
    o_g = gates[:, 3 * _C:4 * _C]
    c_new = jax.nn.sigmoid(f_g) * c_ref[...] + jax.nn.sigmoid(i_g) * jnp.tanh(g_g)
    ctrl = jax.nn.sigmoid(o_g) * jnp.tanh(c_new)
    cnew_ref[...] = c_new
    ctrl_ref[...] = ctrl
    iface_ref[...] = (jnp.dot(ctrl, wif_ref[...], preferred_element_type=jnp.float32)
                      + bif_ref[...])


def _mega_body(mem_ref, link_ref, rw_ref, usage_ref, wwts_ref, prec_ref,
               rk_ref, rbeta_ref, wk_ref, wbeta_ref, ev_ref, wv_ref,
               fg_ref, ag_ref, wg_ref, m0_ref, m1_ref, m2_ref,
               linknew_ref, memnew_ref, rwnew_ref, ww_ref, uu_ref,
               precnew_ref, reads_ref, ucol_s, acol_s, wwcol_s):
    rw = rw_ref[0]
    u = usage_ref[0]
    wwt = wwts_ref[0]

    f_sig = jax.nn.sigmoid(fg_ref[0])
    t = 1.0 - f_sig * rw
    ret = t[0:1]
    for r in range(1, _R):
        ret = ret * t[r:r + 1]
    uu = (u + wwt - u * wwt) * ret
    uu_ref[0] = uu
    ucol_s[...] = jnp.transpose(uu)

    logu = jnp.log(uu)
    ch = 128
    tri = (jax.lax.broadcasted_iota(jnp.int32, (ch, ch), 1)
           < jax.lax.broadcasted_iota(jnp.int32, (ch, ch), 0))
    for s in range(0, _N, ch):
        uu_c = ucol_s[s:s + ch]
        lsum = 0.0
        if s > 0:
            le = uu[:, :s] <= uu_c
            lsum = lsum + jnp.sum(jnp.where(le, logu[:, :s], 0.0),
                                  axis=1, keepdims=True)
        ud = uu[:, s:s + ch]
        cond_d = (ud < uu_c) | ((ud == uu_c) & tri)
        lsum = lsum + jnp.sum(jnp.where(cond_d, logu[:, s:s + ch], 0.0),
                              axis=1, keepdims=True)
        if s + ch < _N:
            lt = uu[:, s + ch:] < uu_c
            lsum = lsum + jnp.sum(jnp.where(lt, logu[:, s + ch:], 0.0),
                                  axis=1, keepdims=True)
        acol_s[s:s + ch] = (1.0 - uu_c) * uu_c * jnp.exp(lsum)
    alloc = jnp.transpose(acol_s[...])

    ones_w = jnp.ones((1, _W), jnp.float32)
    cdims = (((1,), (1,)), ((), ()))
    mem0 = mem_ref[0]
    mem_nsq = jax.lax.dot_general(ones_w, mem0 * mem0, cdims,
                                  preferred_element_type=jnp.float32)
    mem_norm = jnp.sqrt(mem_nsq) + 1e-8
    wk = wk_ref[0]
    wk_n = wk / (jnp.sqrt(jnp.sum(wk * wk, axis=1, keepdims=True)) + 1e-8)
    beta_w = 1.0 + jax.nn.softplus(wbeta_ref[0])
    logits_w = jax.lax.dot_general(wk_n, mem0, cdims,
                                   preferred_element_type=jnp.float32) / mem_norm * beta_w
    logits_w = logits_w - jnp.max(logits_w, axis=1, keepdims=True)
    e_w = jnp.exp(logits_w)
    w_c = e_w / jnp.sum(e_w, axis=1, keepdims=True)

    wg = jax.nn.sigmoid(wg_ref[0])
    ag = jax.nn.sigmoid(ag_ref[0])
    ww = wg * (ag * alloc + (1.0 - ag) * w_c)
    ww_ref[0] = ww
    precnew_ref[0] = (1.0 - jnp.sum(ww, axis=1, keepdims=True)) * prec_ref[0] + ww
    wwcol_s[...] = jnp.transpose(ww)

    ev = jax.nn.sigmoid(ev_ref[0])
    wv = wv_ref[0]
    ww_col = wwcol_s[...]
    memnew_ref[0] = (mem_ref[0] * (1.0 - ww_col * ev) + ww_col * wv)

    offdiag = (jax.lax.broadcasted_iota(jnp.int32, (128, 128), 0)
               != jax.lax.broadcasted_iota(jnp.int32, (128, 128), 1))
    wwr = ww_ref[0]
    prec = prec_ref[0]
    for a in range(0, _N, 128):
        wwc = wwcol_s[a:a + 128]
        cand = ((1.0 - wwc) - wwr) * link_ref[0, a:a + 128, :] + wwc * prec
        linknew_ref[0, a:a + 128, :] = cand
        blk = jnp.where(offdiag, cand[:, a:a + 128], 0.0)
        linknew_ref[0, a:a + 128, a:a + 128] = blk
    lnk = linknew_ref[0]
    fwd = jax.lax.dot_general(rw, lnk, (((1,), (1,)), ((), ())),
                              preferred_element_type=jnp.float32)
    bwd = jax.lax.dot_general(rw, lnk, (((1,), (0,)), ((), ())),
                              preferred_element_type=jnp.float32)

    mn = memnew_ref[0]
    mn_nsq = jax.lax.dot_general(ones_w, mn * mn, cdims,
                                 preferred_element_type=jnp.float32)
    mn_norm = jnp.sqrt(mn_nsq) + 1e-8
    rk = rk_ref[0]
    rk_n = rk / (jnp.sqrt(jnp.sum(rk * rk, axis=1, keepdims=True)) + 1e-8)
    beta_r = 1.0 + jax.nn.softplus(rbeta_ref[0])
    logits_r = jax.lax.dot_general(rk_n, mn, cdims,
                                   preferred_element_type=jnp.float32) / mn_norm * beta_r
    logits_r = logits_r - jnp.max(logits_r, axis=1, keepdims=True)
    e_r = jnp.exp(logits_r)
    r_c = e_r / jnp.sum(e_r, axis=1, keepdims=True)

    m0 = m0_ref[0]
    m1 = m1_ref[0]
    m2 = m2_ref[0]
    mx = jnp.maximum(m0, jnp.maximum(m1, m2))
    e0 = jnp.exp(m0 - mx)
    e1 = jnp.exp(m1 - mx)
    e2 = jnp.exp(m2 - mx)
    es = e0 + e1 + e2
    rwnew = (e0 / es) * bwd + (e1 / es) * r_c + (e2 / es) * fwd
    rwnew_ref[0] = rwnew
    reads_ref[0] = jax.lax.dot_general(rwnew, mn, (((1,), (0,)), ((), ())),
                                       preferred_element_type=jnp.float32)


def _outproj_body(co_ref, wout_ref, bout_ref, out_ref):
    out_ref[...] = (jnp.dot(co_ref[...], wout_ref[...],
                            preferred_element_type=jnp.float32) + bout_ref[...])


def kernel(x, read, h, c, memory, link, r_weights, w_weights, usage,
           precedence, W_ih, W_hh, b_ih, b_hh, W_if, b_if, W_out, b_out):
    B, N, W = memory.shape
    R = r_weights.shape[1]
    C = h.shape[1]

    xr = jnp.concatenate([x, read], axis=1)
    bias = (b_ih + b_hh).reshape(1, 4 * C)
    ctrl, c_new, iface = pl.pallas_call(
        _controller_body,
        out_shape=[
            jax.ShapeDtypeStruct((B, C), jnp.float32),
            jax.ShapeDtypeStruct((B, C), jnp.float32),
            jax.ShapeDtypeStruct((B, _IFACE), jnp.float32),
        ],
        name="dnc_controller",
    )(xr, h, c, W_ih.T, W_hh.T, bias, W_if.T, b_if.reshape(1, _IFACE))

    o = R * W
    r_keys = iface[:, :o].reshape(B, R, W)
    r_betas = iface[:, o:o + R].reshape(B, R, 1); o += R
    w_key = iface[:, o:o + W].reshape(B, 1, W); o += W
    w_beta = iface[:, o:o + 1].reshape(B, 1, 1); o += 1
    e_vec = iface[:, o:o + W].reshape(B, 1, W); o += W
    w_vec = iface[:, o:o + W].reshape(B, 1, W); o += W
    f_gates = iface[:, o:o + R].reshape(B, R, 1); o += R
    a_gate = iface[:, o:o + 1].reshape(B, 1, 1); o += 1
    w_gate = iface[:, o:o + 1].reshape(B, 1, 1); o += 1
    r_modes = iface[:, o:o + 3 * R].reshape(B, R, 3)
    m0 = r_modes[:, :, 0:1]
    m1 = r_modes[:, :, 1:2]
    m2 = r_modes[:, :, 2:3]

    usage3 = usage.reshape(B, 1, N)
    wwts3 = w_weights.reshape(B, 1, N)
    prec3 = precedence.reshape(B, 1, N)

    def _b3(shape):
        return pl.BlockSpec((1,) + shape, lambda b: (b, 0, 0))

    (link_new, mem_new, rw_new, ww3, uu3, precnew3, reads3) = pl.pallas_call(
        _mega_body,
        grid=(B,),
        in_specs=[
            _b3((N, W)),
            _b3((N, N)),
            _b3((R, N)),
            _b3((1, N)),
            _b3((1, N)),
            _b3((1, N)),
            _b3((R, W)),
            _b3((R, 1)),
            _b3((1, W)),
            _b3((1, 1)),
            _b3((1, W)),
            _b3((1, W)),
            _b3((R, 1)),
            _b3((1, 1)),
            _b3((1, 1)),
            _b3((R, 1)),
            _b3((R, 1)),
            _b3((R, 1)),
        ],
        out_specs=[
            _b3((N, N)),
            _b3((N, W)),
            _b3((R, N)),
            _b3((1, N)),
            _b3((1, N)),
            _b3((1, N)),
            _b3((R, W)),
        ],
        out_shape=[
            jax.ShapeDtypeStruct((B, N, N), jnp.float32),
            jax.ShapeDtypeStruct((B, N, W), jnp.float32),
            jax.ShapeDtypeStruct((B, R, N), jnp.float32),
            jax.ShapeDtypeStruct((B, 1, N), jnp.float32),
            jax.ShapeDtypeStruct((B, 1, N), jnp.float32),
            jax.ShapeDtypeStruct((B, 1, N), jnp.float32),
            jax.ShapeDtypeStruct((B, R, W), jnp.float32),
        ],
        scratch_shapes=[
            pltpu.VMEM((_N, 1), jnp.float32),
            pltpu.VMEM((_N, 1), jnp.float32),
            pltpu.VMEM((_N, 1), jnp.float32),
        ],
        compiler_params=pltpu.CompilerParams(
            dimension_semantics=("arbitrary",),
            vmem_limit_bytes=52 * 1024 * 1024,
        ),
        name="dnc_mega",
    )(memory, link, r_weights, usage3, wwts3, prec3, r_keys, r_betas,
      w_key, w_beta, e_vec, w_vec, f_gates, a_gate, w_gate, m0, m1, m2)

    reads_flat = reads3.reshape(B, R * W)

    co = jnp.concatenate([ctrl, reads_flat], axis=1)
    out = pl.pallas_call(
        _outproj_body,
        out_shape=jax.ShapeDtypeStruct((B, _LOUT), jnp.float32),
        name="dnc_outproj",
    )(co, W_out.T, b_out.reshape(1, _LOUT))

    return (out, reads_flat, mem_new, link_new, rw_new,
            ww3.reshape(B, N), uu3.reshape(B, N), precnew3.reshape(B, N),
            ctrl, c_new)


# speedup vs baseline: 1.0520x; 1.0045x over previous
import jax
import jax.numpy as jnp
from jax.experimental import pallas as pl
from jax.experimental.pallas import tpu as pltpu

_B, _N, _W, _R, _C = 64, 1024, 64, 4, 512
_LOUT = 256
_IFACE = _R * _W + _R + _W + 1 + 2 * _W + _R + 2 + 3 * _R


def _controller_body(xr_ref, h_ref, c_ref, wih_ref, whh_ref, bias_ref,
                     wif_ref, bif_ref, ctrl_ref, cnew_ref, iface_ref):
    gates = (jnp.dot(xr_ref[...], wih_ref[...], preferred_element_type=jnp.float32)
             + jnp.dot(h_ref[...], whh_ref[...], preferred_element_type=jnp.float32)
             + bias_ref[...])
    i_g = gates[:, 0 * _C:1 * _C]
    f_g = gates[:, 1 * _C:2 * _C]
    g_g = gates[:, 2 * _C:3 * _C]
    o_g = gates[:, 3 * _C:4 * _C]
    c_new = jax.nn.sigmoid(f_g) * c_ref[...] + jax.nn.sigmoid(i_g) * jnp.tanh(g_g)
    ctrl = jax.nn.sigmoid(o_g) * jnp.tanh(c_new)
    cnew_ref[...] = c_new
    ctrl_ref[...] = ctrl
    iface_ref[...] = (jnp.dot(ctrl, wif_ref[...], preferred_element_type=jnp.float32)
                      + bif_ref[...])


def _mega_body(mem_ref, link_ref, rw_ref, usage_ref, wwts_ref, prec_ref,
               rk_ref, rbeta_ref, wk_ref, wbeta_ref, ev_ref, wv_ref,
               fg_ref, ag_ref, wg_ref, m0_ref, m1_ref, m2_ref,
               linknew_ref, memnew_ref, rwnew_ref, ww_ref, uu_ref,
               precnew_ref, reads_ref, ucol_s, acol_s, wwcol_s):
    rw = rw_ref[0]
    u = usage_ref[0]
    wwt = wwts_ref[0]

    f_sig = jax.nn.sigmoid(fg_ref[0])
    t = 1.0 - f_sig * rw
    ret = t[0:1]
    for r in range(1, _R):
        ret = ret * t[r:r + 1]
    uu = (u + wwt - u * wwt) * ret
    uu_ref[0] = uu
    ucol_s[...] = jnp.transpose(uu)

    logu = jnp.log(uu)
    ch = 128
    tri = (jax.lax.broadcasted_iota(jnp.int32, (ch, ch), 1)
           < jax.lax.broadcasted_iota(jnp.int32, (ch, ch), 0))
    for s in range(0, _N, ch):
        uu_c = ucol_s[s:s + ch]
        lsum = 0.0
        if s > 0:
            le = uu[:, :s] <= uu_c
            lsum = lsum + jnp.sum(jnp.where(le, logu[:, :s], 0.0),
                                  axis=1, keepdims=True)
        ud = uu[:, s:s + ch]
        cond_d = (ud < uu_c) | ((ud == uu_c) & tri)
        lsum = lsum + jnp.sum(jnp.where(cond_d, logu[:, s:s + ch], 0.0),
                              axis=1, keepdims=True)
        if s + ch < _N:
            lt = uu[:, s + ch:] < uu_c
            lsum = lsum + jnp.sum(jnp.where(lt, logu[:, s + ch:], 0.0),
                                  axis=1, keepdims=True)
        acol_s[s:s + ch] = (1.0 - uu_c) * uu_c * jnp.exp(lsum)
    alloc = jnp.transpose(acol_s[...])

    ones_w = jnp.ones((1, _W), jnp.float32)
    cdims = (((1,), (1,)), ((), ()))
    mem0 = mem_ref[0]
    mem_nsq = jax.lax.dot_general(ones_w, mem0 * mem0, cdims,
                                  preferred_element_type=jnp.float32)
    mem_norm = jnp.sqrt(mem_nsq) + 1e-8
    wk = wk_ref[0]
    wk_n = wk / (jnp.sqrt(jnp.sum(wk * wk, axis=1, keepdims=True)) + 1e-8)
    beta_w = 1.0 + jax.nn.softplus(wbeta_ref[0])
    logits_w = jax.lax.dot_general(wk_n, mem0, cdims,
                                   preferred_element_type=jnp.float32) / mem_norm * beta_w
    logits_w = logits_w - jnp.max(logits_w, axis=1, keepdims=True)
    e_w = jnp.exp(logits_w)
    w_c = e_w / jnp.sum(e_w, axis=1, keepdims=True)

    wg = jax.nn.sigmoid(wg_ref[0])
    ag = jax.nn.sigmoid(ag_ref[0])
    ww = wg * (ag * alloc + (1.0 - ag) * w_c)
    ww_ref[0] = ww
    precnew_ref[0] = (1.0 - jnp.sum(ww, axis=1, keepdims=True)) * prec_ref[0] + ww
    wwcol_s[...] = jnp.transpose(ww)

    ev = jax.nn.sigmoid(ev_ref[0])
    wv = wv_ref[0]
    ww_col = wwcol_s[...]
    memnew_ref[0] = (mem_ref[0] * (1.0 - ww_col * ev) + ww_col * wv)

    offdiag = (jax.lax.broadcasted_iota(jnp.int32, (128, 128), 0)
               != jax.lax.broadcasted_iota(jnp.int32, (128, 128), 1))
    wwr = ww_ref[0]
    prec = prec_ref[0]
    c11 = (((1,), (1,)), ((), ()))
    c10 = (((1,), (0,)), ((), ()))
    f32 = jnp.float32
    fwd_parts = []
    bwd = jnp.zeros((_R, _N), f32)
    for a in range(0, _N, 128):
        b_ = a + 128
        wwc = wwcol_s[a:b_]
        cand = ((1.0 - wwc) - wwr) * link_ref[0, a:b_, :] + wwc * prec
        linknew_ref[0, a:b_, :] = cand
        blk = jnp.where(offdiag, cand[:, a:b_], 0.0)
        linknew_ref[0, a:b_, a:b_] = blk
        rwc = rw[:, a:b_]
        fp = jax.lax.dot_general(rwc, blk, c11, preferred_element_type=f32)
        bwd_mid = jax.lax.dot_general(rwc, blk, c10, preferred_element_type=f32)
        bwd_pieces = []
        if a > 0:
            fp = fp + jax.lax.dot_general(rw[:, :a], cand[:, :a], c11,
                                          preferred_element_type=f32)
            bwd_pieces.append(jax.lax.dot_general(rwc, cand[:, :a], c10,
                                                  preferred_element_type=f32))
        bwd_pieces.append(bwd_mid)
        if b_ < _N:
            fp = fp + jax.lax.dot_general(rw[:, b_:], cand[:, b_:], c11,
                                          preferred_element_type=f32)
            bwd_pieces.append(jax.lax.dot_general(rwc, cand[:, b_:], c10,
                                                  preferred_element_type=f32))
        fwd_parts.append(fp)
        bwd = bwd + (jnp.concatenate(bwd_pieces, axis=1)
                     if len(bwd_pieces) > 1 else bwd_pieces[0])
    fwd = jnp.concatenate(fwd_parts, axis=1)

    mn = memnew_ref[0]
    mn_nsq = jax.lax.dot_general(ones_w, mn * mn, cdims,
                                 preferred_element_type=jnp.float32)
    mn_norm = jnp.sqrt(mn_nsq) + 1e-8
    rk = rk_ref[0]
    rk_n = rk / (jnp.sqrt(jnp.sum(rk * rk, axis=1, keepdims=True)) + 1e-8)
    beta_r = 1.0 + jax.nn.softplus(rbeta_ref[0])
    logits_r = jax.lax.dot_general(rk_n, mn, cdims,
                                   preferred_element_type=jnp.float32) / mn_norm * beta_r
    logits_r = logits_r - jnp.max(logits_r, axis=1, keepdims=True)
    e_r = jnp.exp(logits_r)
    r_c = e_r / jnp.sum(e_r, axis=1, keepdims=True)

    m0 = m0_ref[0]
    m1 = m1_ref[0]
    m2 = m2_ref[0]
    mx = jnp.maximum(m0, jnp.maximum(m1, m2))
    e0 = jnp.exp(m0 - mx)
    e1 = jnp.exp(m1 - mx)
    e2 = jnp.exp(m2 - mx)
    es = e0 + e1 + e2
    rwnew = (e0 / es) * bwd + (e1 / es) * r_c + (e2 / es) * fwd
    rwnew_ref[0] = rwnew
    reads_ref[0] = jax.lax.dot_general(rwnew, mn, (((1,), (0,)), ((), ())),
                                       preferred_element_type=jnp.float32)


def _outproj_body(co_ref, wout_ref, bout_ref, out_ref):
    out_ref[...] = (jnp.dot(co_ref[...], wout_ref[...],
                            preferred_element_type=jnp.float32) + bout_ref[...])


def kernel(x, read, h, c, memory, link, r_weights, w_weights, usage,
           precedence, W_ih, W_hh, b_ih, b_hh, W_if, b_if, W_out, b_out):
    B, N, W = memory.shape
    R = r_weights.shape[1]
    C = h.shape[1]

    xr = jnp.concatenate([x, read], axis=1)
    bias = (b_ih + b_hh).reshape(1, 4 * C)
    ctrl, c_new, iface = pl.pallas_call(
        _controller_body,
        out_shape=[
            jax.ShapeDtypeStruct((B, C), jnp.float32),
            jax.ShapeDtypeStruct((B, C), jnp.float32),
            jax.ShapeDtypeStruct((B, _IFACE), jnp.float32),
        ],
        name="dnc_controller",
    )(xr, h, c, W_ih.T, W_hh.T, bias, W_if.T, b_if.reshape(1, _IFACE))

    o = R * W
    r_keys = iface[:, :o].reshape(B, R, W)
    r_betas = iface[:, o:o + R].reshape(B, R, 1); o += R
    w_key = iface[:, o:o + W].reshape(B, 1, W); o += W
    w_beta = iface[:, o:o + 1].reshape(B, 1, 1); o += 1
    e_vec = iface[:, o:o + W].reshape(B, 1, W); o += W
    w_vec = iface[:, o:o + W].reshape(B, 1, W); o += W
    f_gates = iface[:, o:o + R].reshape(B, R, 1); o += R
    a_gate = iface[:, o:o + 1].reshape(B, 1, 1); o += 1
    w_gate = iface[:, o:o + 1].reshape(B, 1, 1); o += 1
    r_modes = iface[:, o:o + 3 * R].reshape(B, R, 3)
    m0 = r_modes[:, :, 0:1]
    m1 = r_modes[:, :, 1:2]
    m2 = r_modes[:, :, 2:3]

    usage3 = usage.reshape(B, 1, N)
    wwts3 = w_weights.reshape(B, 1, N)
    prec3 = precedence.reshape(B, 1, N)

    def _b3(shape):
        return pl.BlockSpec((1,) + shape, lambda b: (b, 0, 0))

    (link_new, mem_new, rw_new, ww3, uu3, precnew3, reads3) = pl.pallas_call(
        _mega_body,
        grid=(B,),
        in_specs=[
            _b3((N, W)),
            _b3((N, N)),
            _b3((R, N)),
            _b3((1, N)),
            _b3((1, N)),
            _b3((1, N)),
            _b3((R, W)),
            _b3((R, 1)),
            _b3((1, W)),
            _b3((1, 1)),
            _b3((1, W)),
            _b3((1, W)),
            _b3((R, 1)),
            _b3((1, 1)),
            _b3((1, 1)),
            _b3((R, 1)),
            _b3((R, 1)),
            _b3((R, 1)),
        ],
        out_specs=[
            _b3((N, N)),
            _b3((N, W)),
            _b3((R, N)),
            _b3((1, N)),
            _b3((1, N)),
            _b3((1, N)),
            _b3((R, W)),
        ],
        out_shape=[
            jax.ShapeDtypeStruct((B, N, N), jnp.float32),
            jax.ShapeDtypeStruct((B, N, W), jnp.float32),
            jax.ShapeDtypeStruct((B, R, N), jnp.float32),
            jax.ShapeDtypeStruct((B, 1, N), jnp.float32),
            jax.ShapeDtypeStruct((B, 1, N), jnp.float32),
            jax.ShapeDtypeStruct((B, 1, N), jnp.float32),
            jax.ShapeDtypeStruct((B, R, W), jnp.float32),
        ],
        scratch_shapes=[
            pltpu.VMEM((_N, 1), jnp.float32),
            pltpu.VMEM((_N, 1), jnp.float32),
            pltpu.VMEM((_N, 1), jnp.float32),
        ],
        compiler_params=pltpu.CompilerParams(
            dimension_semantics=("arbitrary",),
            vmem_limit_bytes=52 * 1024 * 1024,
        ),
        name="dnc_mega",
    )(memory, link, r_weights, usage3, wwts3, prec3, r_keys, r_betas,
      w_key, w_beta, e_vec, w_vec, f_gates, a_gate, w_gate, m0, m1, m2)

    reads_flat = reads3.reshape(B, R * W)

    co = jnp.concatenate([ctrl, reads_flat], axis=1)
    out = pl.pallas_call(
        _outproj_body,
        out_shape=jax.ShapeDtypeStruct((B, _LOUT), jnp.float32),
        name="dnc_outproj",
    )(co, W_out.T, b_out.reshape(1, _LOUT))

    return (out, reads_flat, mem_new, link_new, rw_new,
            ww3.reshape(B, N), uu3.reshape(B, N), precnew3.reshape(B, N),
            ctrl, c_new)


# speedup vs baseline: 1.0663x; 1.0137x over previous
import jax
import jax.numpy as jnp
from jax.experimental import pallas as pl
from jax.experimental.pallas import tpu as pltpu

_B, _N, _W, _R, _C = 64, 1024, 64, 4, 512
_LOUT = 256
_IFACE = _R * _W + _R + _W + 1 + 2 * _W + _R + 2 + 3 * _R


def _controller_body(xr_ref, h_ref, c_ref, wih_ref, whh_ref, bias_ref,
                     wif_ref, bif_ref, ctrl_ref, cnew_ref, iface_ref):
    gates = (jnp.dot(xr_ref[...], wih_ref[...], preferred_element_type=jnp.float32)
             + jnp.dot(h_ref[...], whh_ref[...], preferred_element_type=jnp.float32)
             + bias_ref[...])
    i_g = gates[:, 0 * _C:1 * _C]
    f_g = gates[:, 1 * _C:2 * _C]
    g_g = gates[:, 2 * _C:3 * _C]
    o_g = gates[:, 3 * _C:4 * _C]
    c_new = jax.nn.sigmoid(f_g) * c_ref[...] + jax.nn.sigmoid(i_g) * jnp.tanh(g_g)
    ctrl = jax.nn.sigmoid(o_g) * jnp.tanh(c_new)
    cnew_ref[...] = c_new
    ctrl_ref[...] = ctrl
    iface_ref[...] = (jnp.dot(ctrl, wif_ref[...], preferred_element_type=jnp.float32)
                      + bif_ref[...])


def _mega_body(mem_ref, link_ref, rw_ref, usage_ref, wwts_ref, prec_ref,
               rk_ref, rbeta_ref, wk_ref, wbeta_ref, ev_ref, wv_ref,
               fg_ref, ag_ref, wg_ref, m0_ref, m1_ref, m2_ref,
               linknew_ref, memnew_ref, rwnew_ref, ww_ref, uu_ref,
               precnew_ref, reads_ref):
    mem = mem_ref[0]
    rw = rw_ref[0]
    u = usage_ref[0]
    wwt = wwts_ref[0]
    prec = prec_ref[0]

    f_sig = jax.nn.sigmoid(fg_ref[0])
    t = 1.0 - f_sig * rw
    ret = t[0:1]
    for r in range(1, _R):
        ret = ret * t[r:r + 1]
    uu = (u + wwt - u * wwt) * ret
    uu_ref[0] = uu

    uu_col = jnp.transpose(uu)
    logu = jnp.log(uu)
    ch = 256
    s_parts = []
    iota_j = jax.lax.broadcasted_iota(jnp.int32, (ch, _N), 1)
    iota_i0 = jax.lax.broadcasted_iota(jnp.int32, (ch, _N), 0)
    for s in range(0, _N, ch):
        uu_c = uu_col[s:s + ch]
        lt = uu < uu_c
        tie = (uu == uu_c) & (iota_j < iota_i0 + s)
        lsum = jnp.sum(jnp.where(lt | tie, logu, 0.0), axis=1, keepdims=True)
        s_parts.append(lsum)
    p = jnp.exp(jnp.concatenate(s_parts, axis=0))
    alloc_col = (1.0 - uu_col) * uu_col * p
    alloc = jnp.transpose(alloc_col)

    ones_w = jnp.ones((1, _W), jnp.float32)
    cdims = (((1,), (1,)), ((), ()))
    mem_nsq = jax.lax.dot_general(ones_w, mem * mem, cdims,
                                  preferred_element_type=jnp.float32)
    mem_norm = jnp.sqrt(mem_nsq) + 1e-8
    wk = wk_ref[0]
    wk_n = wk / (jnp.sqrt(jnp.sum(wk * wk, axis=1, keepdims=True)) + 1e-8)
    beta_w = 1.0 + jax.nn.softplus(wbeta_ref[0])
    logits_w = jax.lax.dot_general(wk_n, mem, cdims,
                                   preferred_element_type=jnp.float32) / mem_norm * beta_w
    logits_w = logits_w - jnp.max(logits_w, axis=1, keepdims=True)
    e_w = jnp.exp(logits_w)
    w_c = e_w / jnp.sum(e_w, axis=1, keepdims=True)

    wg = jax.nn.sigmoid(wg_ref[0])
    ag = jax.nn.sigmoid(ag_ref[0])
    ww = wg * (ag * alloc + (1.0 - ag) * w_c)
    ww_ref[0] = ww
    precnew_ref[0] = (1.0 - jnp.sum(ww, axis=1, keepdims=True)) * prec + ww
    ww_col = jnp.transpose(ww)

    link = link_ref[0]
    di = jax.lax.broadcasted_iota(jnp.int32, (_N, _N), 0)
    dj = jax.lax.broadcasted_iota(jnp.int32, (_N, _N), 1)
    linknew = (1.0 - ww_col - ww) * link + ww_col * prec
    linknew = jnp.where(di != dj, linknew, 0.0)
    linknew_ref[0] = linknew
    lnk = linknew_ref[0]
    fwd = jax.lax.dot_general(rw, lnk, (((1,), (1,)), ((), ())),
                              preferred_element_type=jnp.float32)
    bwd = jax.lax.dot_general(rw, lnk, (((1,), (0,)), ((), ())),
                              preferred_element_type=jnp.float32)

    ev = jax.nn.sigmoid(ev_ref[0])
    wv = wv_ref[0]
    memnew = mem * (1.0 - ww_col * ev) + ww_col * wv
    memnew_ref[0] = memnew
    mn = memnew_ref[0]

    mn_nsq = jax.lax.dot_general(ones_w, mn * mn, cdims,
                                 preferred_element_type=jnp.float32)
    mn_norm = jnp.sqrt(mn_nsq) + 1e-8
    rk = rk_ref[0]
    rk_n = rk / (jnp.sqrt(jnp.sum(rk * rk, axis=1, keepdims=True)) + 1e-8)
    beta_r = 1.0 + jax.nn.softplus(rbeta_ref[0])
    logits_r = jax.lax.dot_general(rk_n, mn, cdims,
                                   preferred_element_type=jnp.float32) / mn_norm * beta_r
    logits_r = logits_r - jnp.max(logits_r, axis=1, keepdims=True)
    e_r = jnp.exp(logits_r)
    r_c = e_r / jnp.sum(e_r, axis=1, keepdims=True)

    m0 = m0_ref[0]
    m1 = m1_ref[0]
    m2 = m2_ref[0]
    mx = jnp.maximum(m0, jnp.maximum(m1, m2))
    e0 = jnp.exp(m0 - mx)
    e1 = jnp.exp(m1 - mx)
    e2 = jnp.exp(m2 - mx)
    es = e0 + e1 + e2
    rwnew = (e0 / es) * bwd + (e1 / es) * r_c + (e2 / es) * fwd
    rwnew_ref[0] = rwnew
    reads_ref[0] = jax.lax.dot_general(rwnew, mn, (((1,), (0,)), ((), ())),
                                       preferred_element_type=jnp.float32)


def _outproj_body(co_ref, wout_ref, bout_ref, out_ref):
    out_ref[...] = (jnp.dot(co_ref[...], wout_ref[...],
                            preferred_element_type=jnp.float32) + bout_ref[...])


def kernel(x, read, h, c, memory, link, r_weights, w_weights, usage,
           precedence, W_ih, W_hh, b_ih, b_hh, W_if, b_if, W_out, b_out):
    B, N, W = memory.shape
    R = r_weights.shape[1]
    C = h.shape[1]

    xr = jnp.concatenate([x, read], axis=1)
    bias = (b_ih + b_hh).reshape(1, 4 * C)
    ctrl, c_new, iface = pl.pallas_call(
        _controller_body,
        out_shape=[
            jax.ShapeDtypeStruct((B, C), jnp.float32),
            jax.ShapeDtypeStruct((B, C), jnp.float32),
            jax.ShapeDtypeStruct((B, _IFACE), jnp.float32),
        ],
        name="dnc_controller",
    )(xr, h, c, W_ih.T, W_hh.T, bias, W_if.T, b_if.reshape(1, _IFACE))

    o = R * W
    r_keys = iface[:, :o].reshape(B, R, W)
    r_betas = iface[:, o:o + R].reshape(B, R, 1); o += R
    w_key = iface[:, o:o + W].reshape(B, 1, W); o += W
    w_beta = iface[:, o:o + 1].reshape(B, 1, 1); o += 1
    e_vec = iface[:, o:o + W].reshape(B, 1, W); o += W
    w_vec = iface[:, o:o + W].reshape(B, 1, W); o += W
    f_gates = iface[:, o:o + R].reshape(B, R, 1); o += R
    a_gate = iface[:, o:o + 1].reshape(B, 1, 1); o += 1
    w_gate = iface[:, o:o + 1].reshape(B, 1, 1); o += 1
    r_modes = iface[:, o:o + 3 * R].reshape(B, R, 3)
    m0 = r_modes[:, :, 0:1]
    m1 = r_modes[:, :, 1:2]
    m2 = r_modes[:, :, 2:3]

    usage3 = usage.reshape(B, 1, N)
    wwts3 = w_weights.reshape(B, 1, N)
    prec3 = precedence.reshape(B, 1, N)

    def _b3(shape):
        return pl.BlockSpec((1,) + shape, lambda b: (b, 0, 0))

    (link_new, mem_new, rw_new, ww3, uu3, precnew3, reads3) = pl.pallas_call(
        _mega_body,
        grid=(B,),
        in_specs=[
            _b3((N, W)),
            _b3((N, N)),
            _b3((R, N)),
            _b3((1, N)),
            _b3((1, N)),
            _b3((1, N)),
            _b3((R, W)),
            _b3((R, 1)),
            _b3((1, W)),
            _b3((1, 1)),
            _b3((1, W)),
            _b3((1, W)),
            _b3((R, 1)),
            _b3((1, 1)),
            _b3((1, 1)),
            _b3((R, 1)),
            _b3((R, 1)),
            _b3((R, 1)),
        ],
        out_specs=[
            _b3((N, N)),
            _b3((N, W)),
            _b3((R, N)),
            _b3((1, N)),
            _b3((1, N)),
            _b3((1, N)),
            _b3((R, W)),
        ],
        out_shape=[
            jax.ShapeDtypeStruct((B, N, N), jnp.float32),
            jax.ShapeDtypeStruct((B, N, W), jnp.float32),
            jax.ShapeDtypeStruct((B, R, N), jnp.float32),
            jax.ShapeDtypeStruct((B, 1, N), jnp.float32),
            jax.ShapeDtypeStruct((B, 1, N), jnp.float32),
            jax.ShapeDtypeStruct((B, 1, N), jnp.float32),
            jax.ShapeDtypeStruct((B, R, W), jnp.float32),
        ],
        compiler_params=pltpu.CompilerParams(
            dimension_semantics=("arbitrary",),
            vmem_limit_bytes=52 * 1024 * 1024,
        ),
        name="dnc_mega",
    )(memory, link, r_weights, usage3, wwts3, prec3, r_keys, r_betas,
      w_key, w_beta, e_vec, w_vec, f_gates, a_gate, w_gate, m0, m1, m2)

    reads_flat = reads3.reshape(B, R * W)

    co = jnp.concatenate([ctrl, reads_flat], axis=1)
    out = pl.pallas_call(
        _outproj_body,
        out_shape=jax.ShapeDtypeStruct((B, _LOUT), jnp.float32),
        name="dnc_outproj",
    )(co, W_out.T, b_out.reshape(1, _LOUT))

    return (out, reads_flat, mem_new, link_new, rw_new,
            ww3.reshape(B, N), uu3.reshape(B, N), precnew3.reshape(B, N),
            ctrl, c_new)
